# baseline (speedup 1.0000x reference)
import functools

import jax
import jax.numpy as jnp
from jax import lax
from jax.experimental import pallas as pl
from jax.experimental.pallas import tpu as pltpu
from jax.experimental.pallas import tpu_sc as plsc

DIM = 2048
SEQ_LEN = 8192
TC_ROWS = 6144
SC_ROWS = SEQ_LEN - TC_ROWS
NUM_CORES = 2
NUM_SUBCORES = 16
LANES = 16
NUM_WORKERS = NUM_CORES * NUM_SUBCORES
ROWS_PER_WORKER = SC_ROWS // NUM_WORKERS
CHUNK_ROWS = 32
NUM_CHUNKS = ROWS_PER_WORKER // CHUNK_ROWS
VECS_PER_ROW = DIM // LANES
TC_BLOCK = 512


def _tc_body(emb_ref, out_ref):
    out_ref[...] = emb_ref[...] * (DIM ** -0.5)


def _scale_chunk(buf, scale):
    def row_body(i, _):
        for j in range(VECS_PER_ROW):
            sl = pl.ds(j * LANES, LANES)
            buf[i, sl] = buf[i, sl] * scale
        return 0

    lax.fori_loop(0, CHUNK_ROWS, row_body, 0)


@functools.partial(
    pl.kernel,
    out_type=jax.ShapeDtypeStruct((SC_ROWS, DIM), jnp.float32),
    mesh=plsc.VectorSubcoreMesh(core_axis_name="c", subcore_axis_name="s"),
    scratch_types=[pltpu.VMEM((CHUNK_ROWS, DIM), jnp.float32)],
)
def _pos_emb_sc(emb_hbm, out_hbm, buf):
    scale = jnp.float32(DIM ** -0.5)
    wid = lax.axis_index("s") * NUM_CORES + lax.axis_index("c")
    base = wid * ROWS_PER_WORKER
    for k in range(NUM_CHUNKS):
        row0 = base + k * CHUNK_ROWS
        pltpu.sync_copy(emb_hbm.at[pl.ds(TC_ROWS + row0, CHUNK_ROWS)], buf)
        _scale_chunk(buf, scale)
        pltpu.sync_copy(buf, out_hbm.at[pl.ds(row0, CHUNK_ROWS)])


def kernel(x, emb):
    sc_part = _pos_emb_sc(emb)
    tc_part = pl.pallas_call(
        _tc_body,
        out_shape=jax.ShapeDtypeStruct((TC_ROWS, DIM), jnp.float32),
        grid=(TC_ROWS // TC_BLOCK,),
        in_specs=[pl.BlockSpec((TC_BLOCK, DIM), lambda i: (i, 0))],
        out_specs=pl.BlockSpec((TC_BLOCK, DIM), lambda i: (i, 0)),
    )(emb[:TC_ROWS])
    return jnp.concatenate([tc_part, sc_part], axis=0)


# speedup vs baseline: 1.4055x; 1.4055x over previous
import functools

import jax
import jax.numpy as jnp
from jax import lax
from jax.experimental import pallas as pl
from jax.experimental.pallas import tpu as pltpu
from jax.experimental.pallas import tpu_sc as plsc

DIM = 2048
SEQ_LEN = 8192
NUM_CORES = 2
NUM_SUBCORES = 16
LANES = 16
NUM_WORKERS = NUM_CORES * NUM_SUBCORES
ROWS_PER_WORKER = SEQ_LEN // NUM_WORKERS
SLOT_ROWS = 16
NUM_SLOTS = ROWS_PER_WORKER // SLOT_ROWS
HALF_ROWS = SLOT_ROWS // 2
VECS_PER_ROW = DIM // LANES


def _scale_half(src, src_row0, dst, scale):
    def row_body(i, _):
        for j in range(VECS_PER_ROW):
            sl = pl.ds(j * LANES, LANES)
            dst[i, sl] = src[src_row0 + i, sl] * scale
        return 0

    lax.fori_loop(0, HALF_ROWS, row_body, 0)


@functools.partial(
    pl.kernel,
    out_type=jax.ShapeDtypeStruct((SEQ_LEN, DIM), jnp.float32),
    mesh=plsc.VectorSubcoreMesh(core_axis_name="c", subcore_axis_name="s"),
    scratch_types=(
        [pltpu.VMEM((SLOT_ROWS, DIM), jnp.float32)] * 2
        + [pltpu.VMEM((HALF_ROWS, DIM), jnp.float32)] * 2
        + [pltpu.SemaphoreType.DMA] * 4
    ),
)
def _pos_emb_sc(emb_hbm, out_hbm, in0, in1, st0, st1, isem0, isem1, osem0, osem1):
    scale = jnp.float32(DIM ** -0.5)
    in_bufs = (in0, in1)
    in_sems = (isem0, isem1)
    out_bufs = (st0, st1)
    out_sems = (osem0, osem1)
    wid = lax.axis_index("s") * NUM_CORES + lax.axis_index("c")
    base = wid * ROWS_PER_WORKER

    def in_slice(k):
        return emb_hbm.at[pl.ds(base + k * SLOT_ROWS, SLOT_ROWS)]

    def out_half_slice(k, h):
        return out_hbm.at[pl.ds(base + k * SLOT_ROWS + h * HALF_ROWS, HALF_ROWS)]

    def slot(k, b, first):
        pltpu.make_async_copy(in_slice(k), in_bufs[b], in_sems[b]).wait()
        for h in range(2):
            if not first:
                pltpu.make_async_copy(
                    out_bufs[h], out_half_slice(k - 1, h), out_sems[h]
                ).wait()
            _scale_half(in_bufs[b], h * HALF_ROWS, out_bufs[h], scale)
            pltpu.async_copy(out_bufs[h], out_half_slice(k, h), out_sems[h])

    pltpu.async_copy(in_slice(0), in_bufs[0], in_sems[0])
    pltpu.async_copy(in_slice(1), in_bufs[1], in_sems[1])
    slot(0, 0, True)
    pltpu.async_copy(in_slice(2), in_bufs[0], in_sems[0])
    slot(1, 1, False)
    pltpu.async_copy(in_slice(3), in_bufs[1], in_sems[1])

    @pl.loop(1, NUM_SLOTS // 2)
    def _group(g):
        for b in range(2):
            k = 2 * g + b
            slot(k, b, False)

            @pl.when(k + 2 < NUM_SLOTS)
            def _():
                pltpu.async_copy(in_slice(k + 2), in_bufs[b], in_sems[b])

    for h in range(2):
        pltpu.make_async_copy(
            out_bufs[h], out_half_slice(NUM_SLOTS - 1, h), out_sems[h]
        ).wait()


def kernel(x, emb):
    seq_len = x.shape[1]
    assert seq_len == SEQ_LEN
    return _pos_emb_sc(emb)
